# baseline (speedup 1.0000x reference)
import functools

import jax
import jax.numpy as jnp
from jax import lax
from jax.experimental import pallas as pl
from jax.experimental.pallas import tpu as pltpu
from jax.experimental.pallas import tpu_sc as plsc

D = 32
NC = 2
NS = 16
NW = NC * NS
CHUNK = 800
NBUF = 4
NSPLIT = 4


@functools.partial(jax.jit, static_argnames=("n_chunks",))
def _sc_gather(idx, table, n_chunks):
    b_per_w = n_chunks * CHUNK
    n_idx = NW * b_per_w
    n_rounds = n_chunks // NBUF
    mesh = plsc.VectorSubcoreMesh(core_axis_name="c", subcore_axis_name="s")

    @functools.partial(
        pl.kernel,
        mesh=mesh,
        out_type=jax.ShapeDtypeStruct((n_idx, D), jnp.float32),
        scratch_types=[
            pltpu.VMEM((b_per_w,), jnp.int32),
            [pltpu.VMEM((CHUNK, D), jnp.float32) for _ in range(NBUF)],
            [pltpu.SemaphoreType.DMA for _ in range(NBUF)],
            [pltpu.SemaphoreType.DMA for _ in range(NBUF)],
        ],
        compiler_params=pltpu.CompilerParams(use_tc_tiling_on_sc=False),
    )
    def k(idx_hbm, table_hbm, out_hbm, idx_v, rows, gsems, ssems):
        wid = lax.axis_index("s") * NC + lax.axis_index("c")
        base = wid * b_per_w
        pltpu.sync_copy(idx_hbm.at[wid], idx_v)

        def gather(j, buf, sem):
            return pltpu.async_copy(
                table_hbm.at[idx_v.at[pl.ds(j * CHUNK, CHUNK)]], buf, sem)

        def store(j, buf, sem):
            pltpu.async_copy(
                buf, out_hbm.at[pl.ds(base + j * CHUNK, CHUNK)], sem)

        def store_wait(j, buf, sem):
            pltpu.make_async_copy(
                buf, out_hbm.at[pl.ds(base + j * CHUNK, CHUNK)], sem).wait()

        def body(t, carry):
            j = NBUF * t
            handles = []
            for b in range(NBUF):

                @pl.when(t > 0)
                def _(b=b):
                    store_wait(j - NBUF + b, rows[b], ssems[b])

                handles.append(gather(j + b, rows[b], gsems[b]))
            for b in range(NBUF):
                handles[b].wait()
                store(j + b, rows[b], ssems[b])
            return carry

        lax.fori_loop(0, n_rounds, body, 0)

        def drain(t, carry):
            j = NBUF * t
            for b in range(NBUF):
                store_wait(j + b, rows[b], ssems[b])
            return carry

        lax.fori_loop(n_rounds - 1, n_rounds, drain, 0)

    return k(idx, table)


def kernel(utts, embedding_weight):
    B, H = utts.shape
    part_rows = B // NSPLIT
    n_chunks = part_rows * H // (NW * CHUNK)
    outs = []
    for p in range(NSPLIT):
        part = utts[p * part_rows:(p + 1) * part_rows]
        idx = part.reshape(NW, n_chunks * CHUNK)
        out = _sc_gather(idx, embedding_weight, n_chunks)
        outs.append(out.reshape(part_rows, H, D))
    return jnp.concatenate(outs, axis=0)


# speedup vs baseline: 1.0012x; 1.0012x over previous
import functools

import jax
import jax.numpy as jnp
from jax import lax
from jax.experimental import pallas as pl
from jax.experimental.pallas import tpu as pltpu
from jax.experimental.pallas import tpu_sc as plsc

D = 32
NC = 2
NS = 16
NW = NC * NS
CHUNK = 1600
NBUF = 2
NSPLIT = 4


@functools.partial(jax.jit, static_argnames=("n_chunks",))
def _sc_gather(idx, table, n_chunks):
    b_per_w = n_chunks * CHUNK
    n_idx = NW * b_per_w
    n_rounds = n_chunks // NBUF
    mesh = plsc.VectorSubcoreMesh(core_axis_name="c", subcore_axis_name="s")

    @functools.partial(
        pl.kernel,
        mesh=mesh,
        out_type=jax.ShapeDtypeStruct((n_idx, D), jnp.float32),
        scratch_types=[
            pltpu.VMEM((b_per_w,), jnp.int32),
            [pltpu.VMEM((CHUNK, D), jnp.float32) for _ in range(NBUF)],
            [pltpu.SemaphoreType.DMA for _ in range(NBUF)],
            [pltpu.SemaphoreType.DMA for _ in range(NBUF)],
        ],
        compiler_params=pltpu.CompilerParams(use_tc_tiling_on_sc=False),
    )
    def k(idx_hbm, table_hbm, out_hbm, idx_v, rows, gsems, ssems):
        wid = lax.axis_index("s") * NC + lax.axis_index("c")
        base = wid * b_per_w
        pltpu.sync_copy(idx_hbm.at[wid], idx_v)

        def gather(j, buf, sem):
            return pltpu.async_copy(
                table_hbm.at[idx_v.at[pl.ds(j * CHUNK, CHUNK)]], buf, sem)

        def store(j, buf, sem):
            pltpu.async_copy(
                buf, out_hbm.at[pl.ds(base + j * CHUNK, CHUNK)], sem)

        def store_wait(j, buf, sem):
            pltpu.make_async_copy(
                buf, out_hbm.at[pl.ds(base + j * CHUNK, CHUNK)], sem).wait()

        def body(t, carry):
            j = NBUF * t
            handles = []
            for b in range(NBUF):

                @pl.when(t > 0)
                def _(b=b):
                    store_wait(j - NBUF + b, rows[b], ssems[b])

                handles.append(gather(j + b, rows[b], gsems[b]))
            for b in range(NBUF):
                handles[b].wait()
                store(j + b, rows[b], ssems[b])
            return carry

        lax.fori_loop(0, n_rounds, body, 0)

        def drain(t, carry):
            j = NBUF * t
            for b in range(NBUF):
                store_wait(j + b, rows[b], ssems[b])
            return carry

        lax.fori_loop(n_rounds - 1, n_rounds, drain, 0)

    return k(idx, table)


def kernel(utts, embedding_weight):
    B, H = utts.shape
    part_rows = B // NSPLIT
    n_chunks = part_rows * H // (NW * CHUNK)
    outs = []
    for p in range(NSPLIT):
        part = utts[p * part_rows:(p + 1) * part_rows]
        idx = part.reshape(NW, n_chunks * CHUNK)
        out = _sc_gather(idx, embedding_weight, n_chunks)
        outs.append(out.reshape(part_rows, H, D))
    return jnp.concatenate(outs, axis=0)
